# baseline (speedup 1.0000x reference)
import jax
import jax.numpy as jnp
from jax.experimental import pallas as pl


def _copy_body(x_ref, o_ref):
    o_ref[...] = x_ref[...]


def kernel(scores):
    B, T, T_kv = scores.shape
    k_effective = min(2048, 4096, T_kv)
    s = pl.pallas_call(
        _copy_body,
        grid=(B,),
        in_specs=[pl.BlockSpec((1, T, T_kv), lambda i: (i, 0, 0))],
        out_specs=pl.BlockSpec((1, T, T_kv), lambda i: (i, 0, 0)),
        out_shape=jax.ShapeDtypeStruct(scores.shape, scores.dtype),
    )(scores)
    scores_for_topk = jnp.where(s == -jnp.inf, jnp.float32(-1e9), s)
    _, indices = jax.lax.top_k(scores_for_topk, k_effective)
    gathered = jnp.take_along_axis(s, indices, axis=-1)
    mask = gathered != -jnp.inf
    return indices, mask, jnp.array(k_effective, dtype=jnp.int32)


# speedup vs baseline: 6.3696x; 6.3696x over previous
import functools

import jax
import jax.numpy as jnp
from jax import lax
from jax.experimental import pallas as pl
from jax.experimental.pallas import tpu as pltpu
from jax.experimental.pallas import tpu_sc as plsc

L = 16
K = 2048
NB = 2048
CAP = 4096
T_KV = 32768
N_ROWS = 512
N_WORKERS = 32
ROWS_PER_W = N_ROWS // N_WORKERS
NV_DATA = T_KV // L


def _iota():
    return lax.iota(jnp.int32, L)


def _keys_from_f32(v):
    bits = lax.bitcast_convert_type(v, jnp.int32)
    return bits ^ (lax.shift_right_arithmetic(bits, 31) & jnp.int32(0x7FFFFFFF))


def _vals_from_keys(k):
    bits = k ^ (lax.shift_right_arithmetic(k, 31) & jnp.int32(0x7FFFFFFF))
    return lax.bitcast_convert_type(bits, jnp.float32)


def _vgather(x, idx):
    return lax.gather(
        x, idx[:, None],
        dimension_numbers=lax.GatherDimensionNumbers(
            offset_dims=(), collapsed_slice_dims=(0,), start_index_map=(0,)),
        slice_sizes=(1,), mode=lax.GatherScatterMode.PROMISE_IN_BOUNDS)


def _scalar(x):
    return lax.reduce_max(x, (0,)) if x.ndim else x


def _ds16(j):
    return pl.ds(pl.multiple_of(j * L, L), L)


def _top_digit(key):
    return lax.shift_right_arithmetic(key, 21) + jnp.int32(1024)


def _sc_body(scores_hbm, idx_hbm, val_hbm, data, hist, ck_a, ci_a, ck_b, ci_b,
             offs, stage_v):
    cid = lax.axis_index("c")
    sid = lax.axis_index("s")
    wid = sid * 2 + cid
    iota = _iota()
    lane_base = iota * NB
    ones = jnp.ones((L,), jnp.int32)
    zeros = jnp.zeros((L,), jnp.int32)

    def row_body(t, _carry):
        row = wid * ROWS_PER_W + t
        pltpu.sync_copy(scores_hbm.at[row], data)

        def zero_hist(j, _):
            hist[_ds16(j)] = zeros
            return 0
        lax.fori_loop(0, (L * NB) // L, zero_hist, 0)

        def hist_step(j, _):
            v = data[_ds16(j)]
            d = _top_digit(_keys_from_f32(v))
            plsc.addupdate_scatter(hist, [lane_base + d], ones)
            return 0
        lax.fori_loop(0, NV_DATA, hist_step, 0)

        def sel_step(i, carry):
            bkt, run, found = carry
            c = (NB // L - 1) - i
            tot = zeros
            for l in range(L):
                tot = tot + hist[pl.ds(pl.multiple_of(l * NB, L) + pl.multiple_of(c * L, L), L)]
            rv = lax.rev(tot, (0,))
            cs = plsc.cumsum(rv) + run
            ge = cs >= K
            any_ = jnp.any(ge)
            j = _scalar(plsc.all_reduce_ffs(ge))
            bcand = c * L + (L - 1) - j
            bkt = jnp.where(jnp.logical_or(found, jnp.logical_not(any_)), bkt, bcand)
            found = jnp.logical_or(found, any_)
            run = _scalar(cs)
            return bkt, run, found
        bkt, _, _ = lax.fori_loop(
            0, NB // L, sel_step,
            (jnp.int32(0), jnp.int32(0), jnp.bool_(False)))

        def compact_step(j, off):
            v = data[_ds16(j)]
            key = _keys_from_f32(v)
            d = _top_digit(key)
            sel = d >= bkt
            pc = plsc.cumsum(sel.astype(jnp.int32))
            pos = off + pc - 1
            ok = jnp.logical_and(sel, pos < CAP)
            plsc.store_scatter(ck_a, [pos], key, mask=ok)
            plsc.store_scatter(ci_a, [pos], iota + j * L, mask=ok)
            return off + plsc.all_reduce_population_count(sel)
        offv = lax.fori_loop(0, NV_DATA, compact_step, zeros)
        n_cand = jnp.minimum(_scalar(offv), jnp.int32(CAP))
        pp = n_cand + iota
        pm = pp < CAP
        plsc.store_scatter(ck_a, [pp], jnp.full((L,), -0x80000000, jnp.int32),
                           mask=pm)
        plsc.store_scatter(ci_a, [pp], zeros, mask=pm)
        nv = (n_cand + (L - 1)) // L

        for p, (shift, nbp, top) in enumerate(((0, 2048, False),
                                               (11, 2048, False),
                                               (22, 1024, True))):
            src_k, src_i = (ck_a, ci_a) if p % 2 == 0 else (ck_b, ci_b)
            dst_k, dst_i = (ck_b, ci_b) if p % 2 == 0 else (ck_a, ci_a)

            def digit(key):
                if top:
                    return lax.shift_right_arithmetic(key, shift) + jnp.int32(nbp // 2)
                return lax.shift_right_logical(key, shift) & jnp.int32(0x7FF)

            def zero_offs(j, _):
                offs[_ds16(j)] = zeros
                return 0
            lax.fori_loop(0, nbp // L, zero_offs, 0)

            def sort_hist_step(j, _):
                key = src_k[_ds16(j)]
                d = digit(key)
                comp = (d * L) | iota
                srt = lax.sort(comp)
                sd = lax.shift_right_logical(srt, 4)
                nxt = _vgather(sd, jnp.minimum(iota + 1, L - 1))
                lm = jnp.logical_or(iota == L - 1, sd != nxt)
                prev = _vgather(sd, jnp.maximum(iota - 1, 0))
                bnd = jnp.logical_or(iota == 0, sd != prev)
                start = plsc.cummax(jnp.where(bnd, iota, 0))
                cnt = iota - start + 1
                plsc.addupdate_scatter(offs, [sd], cnt, mask=lm)
                return 0
            lax.fori_loop(0, nv, sort_hist_step, 0)

            def suffix_step(i, run):
                c = (nbp // L - 1) - i
                v = offs[_ds16(c)]
                rv = lax.rev(v, (0,))
                cs = plsc.cumsum(rv)
                ex = (cs - rv) + run
                offs[_ds16(c)] = lax.rev(ex, (0,))
                return run + _scalar(cs)
            lax.fori_loop(0, nbp // L, suffix_step, jnp.int32(0))

            def permute_step(j, _):
                key = src_k[_ds16(j)]
                idxv = src_i[_ds16(j)]
                d = digit(key)
                comp = (d * L) | iota
                srt = lax.sort(comp)
                sd = lax.shift_right_logical(srt, 4)
                sl = srt & jnp.int32(L - 1)
                prev = _vgather(sd, jnp.maximum(iota - 1, 0))
                bnd = jnp.logical_or(iota == 0, sd != prev)
                start = plsc.cummax(jnp.where(bnd, iota, 0))
                rank = iota - start
                base = plsc.load_gather(offs, [sd])
                pos = base + rank
                plsc.store_scatter(dst_k, [pos], _vgather(key, sl))
                plsc.store_scatter(dst_i, [pos], _vgather(idxv, sl))
                nxt = _vgather(sd, jnp.minimum(iota + 1, L - 1))
                lm = jnp.logical_or(iota == L - 1, sd != nxt)
                plsc.addupdate_scatter(offs, [sd], rank + 1, mask=lm)
                return 0
            lax.fori_loop(0, nv, permute_step, 0)

        def out_step(j, _):
            key = ck_b[_ds16(j)]
            stage_v[_ds16(j)] = _vals_from_keys(key)
            return 0
        lax.fori_loop(0, K // L, out_step, 0)
        pltpu.sync_copy(ci_b.at[pl.ds(0, K)], idx_hbm.at[row])
        pltpu.sync_copy(stage_v, val_hbm.at[row])
        return 0

    lax.fori_loop(0, ROWS_PER_W, row_body, 0)


@jax.jit
def _topk_sc(flat_scores):
    mesh = plsc.VectorSubcoreMesh(core_axis_name="c", subcore_axis_name="s")
    f = pl.kernel(
        _sc_body,
        out_type=(
            jax.ShapeDtypeStruct((N_ROWS, K), jnp.int32),
            jax.ShapeDtypeStruct((N_ROWS, K), jnp.float32),
        ),
        mesh=mesh,
        compiler_params=pltpu.CompilerParams(needs_layout_passes=False),
        scratch_types=[
            pltpu.VMEM((T_KV,), jnp.float32),
            pltpu.VMEM((L * NB,), jnp.int32),
            pltpu.VMEM((CAP,), jnp.int32),
            pltpu.VMEM((CAP,), jnp.int32),
            pltpu.VMEM((CAP,), jnp.int32),
            pltpu.VMEM((CAP,), jnp.int32),
            pltpu.VMEM((NB,), jnp.int32),
            pltpu.VMEM((K,), jnp.float32),
        ],
    )
    return f(flat_scores)


def kernel(scores):
    B, T, T_kv = scores.shape
    flat = scores.reshape(B * T, T_kv)
    idx, vals = _topk_sc(flat)
    indices = idx.reshape(B, T, K)
    mask = (vals != -jnp.inf).reshape(B, T, K)
    return indices, mask, jnp.array(K, dtype=jnp.int32)


# speedup vs baseline: 7.6410x; 1.1996x over previous
import functools

import jax
import jax.numpy as jnp
from jax import lax
from jax.experimental import pallas as pl
from jax.experimental.pallas import tpu as pltpu
from jax.experimental.pallas import tpu_sc as plsc

L = 16
K = 2048
NB = 2048
CAP = 4096
T_KV = 32768
N_ROWS = 512
N_WORKERS = 32
ROWS_PER_W = N_ROWS // N_WORKERS
NV_DATA = T_KV // L


def _iota():
    return lax.iota(jnp.int32, L)


def _keys_from_f32(v):
    bits = lax.bitcast_convert_type(v, jnp.int32)
    return bits ^ (lax.shift_right_arithmetic(bits, 31) & jnp.int32(0x7FFFFFFF))


def _vals_from_keys(k):
    bits = k ^ (lax.shift_right_arithmetic(k, 31) & jnp.int32(0x7FFFFFFF))
    return lax.bitcast_convert_type(bits, jnp.float32)


def _vgather(x, idx):
    return lax.gather(
        x, idx[:, None],
        dimension_numbers=lax.GatherDimensionNumbers(
            offset_dims=(), collapsed_slice_dims=(0,), start_index_map=(0,)),
        slice_sizes=(1,), mode=lax.GatherScatterMode.PROMISE_IN_BOUNDS)


def _scalar(x):
    return lax.reduce_max(x, (0,)) if x.ndim else x


def _ds16(j):
    return pl.ds(pl.multiple_of(j * L, L), L)


def _top_digit(key):
    return lax.shift_right_arithmetic(key, 21) + jnp.int32(1024)


def _sc_body(scores_hbm, idx_hbm, val_hbm, data, hist, ck_a, ci_a, ck_b, ci_b,
             offs, stage_v):
    cid = lax.axis_index("c")
    sid = lax.axis_index("s")
    wid = sid * 2 + cid
    iota = _iota()
    lane_base = iota * NB + 1024
    ones = jnp.ones((L,), jnp.int32)
    zeros = jnp.zeros((L,), jnp.int32)

    def row_body(t, _carry):
        row = wid * ROWS_PER_W + t
        pltpu.sync_copy(scores_hbm.at[row], data)

        def zero_hist(j, _):
            hist[_ds16(j)] = zeros
            return 0
        lax.fori_loop(0, (L * NB) // L, zero_hist, 0, unroll=8)

        def hist_step(j, _):
            v = data[_ds16(j)]
            d = lax.shift_right_arithmetic(_keys_from_f32(v), 21)
            plsc.addupdate_scatter(hist, [lane_base + d], ones)
            return 0
        lax.fori_loop(0, NV_DATA, hist_step, 0, unroll=4)

        def sel_step(i, carry):
            bkt, run, found = carry
            c = (NB // L - 1) - i
            tot = zeros
            for l in range(L):
                tot = tot + hist[pl.ds(pl.multiple_of(l * NB, L) + pl.multiple_of(c * L, L), L)]
            rv = lax.rev(tot, (0,))
            cs = plsc.cumsum(rv) + run
            ge = cs >= K
            any_ = jnp.any(ge)
            j = _scalar(plsc.all_reduce_ffs(ge))
            bcand = c * L + (L - 1) - j
            bkt = jnp.where(jnp.logical_or(found, jnp.logical_not(any_)), bkt, bcand)
            found = jnp.logical_or(found, any_)
            run = _scalar(cs)
            return bkt, run, found
        bkt, _, _ = lax.fori_loop(
            0, NB // L, sel_step,
            (jnp.int32(0), jnp.int32(0), jnp.bool_(False)), unroll=2)

        kthr = lax.shift_left(bkt - jnp.int32(1024), 21)

        def compact_step(j, carry):
            offm1, idxv = carry
            v = data[_ds16(j)]
            key = _keys_from_f32(v)
            sel = key >= kthr
            pc = plsc.cumsum(sel.astype(jnp.int32))
            pos = offm1 + pc
            ok = jnp.logical_and(sel, pos < CAP)
            plsc.store_scatter(ck_a, [pos], key, mask=ok)
            plsc.store_scatter(ci_a, [pos], idxv, mask=ok)
            return (offm1 + plsc.all_reduce_population_count(sel),
                    idxv + jnp.int32(L))
        offm1, _ = lax.fori_loop(0, NV_DATA, compact_step,
                                 (zeros - 1, iota), unroll=4)
        n_cand = jnp.minimum(_scalar(offm1) + 1, jnp.int32(CAP))
        minkey = jnp.full((L,), -0x80000000, jnp.int32)
        for m in range(4):
            pp = n_cand + iota + (m * L)
            plsc.store_scatter(ck_a, [pp], minkey, mask=pp < CAP)
            plsc.store_scatter(ci_a, [pp], zeros, mask=pp < CAP)
        ng = (n_cand + (4 * L - 1)) // (4 * L)

        for p, (shift, nbp, top) in enumerate(((0, 2048, False),
                                               (11, 2048, False),
                                               (22, 1024, True))):
            src_k, src_i = (ck_a, ci_a) if p % 2 == 0 else (ck_b, ci_b)
            dst_k, dst_i = (ck_b, ci_b) if p % 2 == 0 else (ck_a, ci_a)

            def digit(key):
                if top:
                    return lax.shift_right_arithmetic(key, shift) + jnp.int32(nbp // 2)
                return lax.shift_right_logical(key, shift) & jnp.int32(0x7FF)

            def zero_offs(j, _):
                offs[_ds16(j)] = zeros
                return 0
            lax.fori_loop(0, nbp // L, zero_offs, 0, unroll=8)

            def sort_hist_step(g, _):
                for u in range(4):
                    j = g * 4 + u
                    key = src_k[_ds16(j)]
                    d = digit(key)
                    comp = (d * L) | iota
                    srt = lax.sort(comp)
                    sd = lax.shift_right_logical(srt, 4)
                    nxt = _vgather(sd, jnp.minimum(iota + 1, L - 1))
                    lm = jnp.logical_or(iota == L - 1, sd != nxt)
                    prev = _vgather(sd, jnp.maximum(iota - 1, 0))
                    bnd = jnp.logical_or(iota == 0, sd != prev)
                    start = plsc.cummax(jnp.where(bnd, iota, 0))
                    cnt = iota - start + 1
                    plsc.addupdate_scatter(offs, [sd], cnt, mask=lm)
                return 0
            lax.fori_loop(0, ng, sort_hist_step, 0)

            def suffix_step(i, run):
                c = (nbp // L - 1) - i
                v = offs[_ds16(c)]
                rv = lax.rev(v, (0,))
                cs = plsc.cumsum(rv)
                ex = (cs - rv) + run
                offs[_ds16(c)] = lax.rev(ex, (0,))
                return run + _scalar(cs)
            lax.fori_loop(0, nbp // L, suffix_step, jnp.int32(0), unroll=2)

            def permute_step(g, _):
                for u in range(4):
                    j = g * 4 + u
                    key = src_k[_ds16(j)]
                    idxv = src_i[_ds16(j)]
                    d = digit(key)
                    comp = (d * L) | iota
                    srt = lax.sort(comp)
                    sd = lax.shift_right_logical(srt, 4)
                    sl = srt & jnp.int32(L - 1)
                    prev = _vgather(sd, jnp.maximum(iota - 1, 0))
                    bnd = jnp.logical_or(iota == 0, sd != prev)
                    start = plsc.cummax(jnp.where(bnd, iota, 0))
                    rank = iota - start
                    base = plsc.load_gather(offs, [sd])
                    pos = base + rank
                    plsc.store_scatter(dst_k, [pos], _vgather(key, sl))
                    plsc.store_scatter(dst_i, [pos], _vgather(idxv, sl))
                    nxt = _vgather(sd, jnp.minimum(iota + 1, L - 1))
                    lm = jnp.logical_or(iota == L - 1, sd != nxt)
                    plsc.addupdate_scatter(offs, [sd], rank + 1, mask=lm)
                return 0
            lax.fori_loop(0, ng, permute_step, 0)

        def out_step(j, _):
            key = ck_b[_ds16(j)]
            stage_v[_ds16(j)] = _vals_from_keys(key)
            return 0
        lax.fori_loop(0, K // L, out_step, 0, unroll=4)
        pltpu.sync_copy(ci_b.at[pl.ds(0, K)], idx_hbm.at[row])
        pltpu.sync_copy(stage_v, val_hbm.at[row])
        return 0

    lax.fori_loop(0, ROWS_PER_W, row_body, 0)


@jax.jit
def _topk_sc(flat_scores):
    mesh = plsc.VectorSubcoreMesh(core_axis_name="c", subcore_axis_name="s")
    f = pl.kernel(
        _sc_body,
        out_type=(
            jax.ShapeDtypeStruct((N_ROWS, K), jnp.int32),
            jax.ShapeDtypeStruct((N_ROWS, K), jnp.float32),
        ),
        mesh=mesh,
        compiler_params=pltpu.CompilerParams(needs_layout_passes=False),
        scratch_types=[
            pltpu.VMEM((T_KV,), jnp.float32),
            pltpu.VMEM((L * NB,), jnp.int32),
            pltpu.VMEM((CAP,), jnp.int32),
            pltpu.VMEM((CAP,), jnp.int32),
            pltpu.VMEM((CAP,), jnp.int32),
            pltpu.VMEM((CAP,), jnp.int32),
            pltpu.VMEM((NB,), jnp.int32),
            pltpu.VMEM((K,), jnp.float32),
        ],
    )
    return f(flat_scores)


def kernel(scores):
    B, T, T_kv = scores.shape
    flat = scores.reshape(B * T, T_kv)
    idx, vals = _topk_sc(flat)
    indices = idx.reshape(B, T, K)
    mask = (vals != -jnp.inf).reshape(B, T, K)
    return indices, mask, jnp.array(K, dtype=jnp.int32)


# speedup vs baseline: 9.7938x; 1.2817x over previous
import functools

import jax
import jax.numpy as jnp
from jax import lax
from jax.experimental import pallas as pl
from jax.experimental.pallas import tpu as pltpu
from jax.experimental.pallas import tpu_sc as plsc

L = 16
K = 2048
NB = 2048
CAP = 4096
T_KV = 32768
N_ROWS = 512
N_WORKERS = 32
ROWS_PER_W = N_ROWS // N_WORKERS
NV_DATA = T_KV // L
NV_SAMPLE = 128
M_SAMPLE = 168


def _iota():
    return lax.iota(jnp.int32, L)


def _keys_from_f32(v):
    bits = lax.bitcast_convert_type(v, jnp.int32)
    return bits ^ (lax.shift_right_arithmetic(bits, 31) & jnp.int32(0x7FFFFFFF))


def _vals_from_keys(k):
    bits = k ^ (lax.shift_right_arithmetic(k, 31) & jnp.int32(0x7FFFFFFF))
    return lax.bitcast_convert_type(bits, jnp.float32)


def _vgather(x, idx):
    return lax.gather(
        x, idx[:, None],
        dimension_numbers=lax.GatherDimensionNumbers(
            offset_dims=(), collapsed_slice_dims=(0,), start_index_map=(0,)),
        slice_sizes=(1,), mode=lax.GatherScatterMode.PROMISE_IN_BOUNDS)


def _splat_last(x):
    return _vgather(x, jnp.full((L,), L - 1, jnp.int32))


def _scalar(x):
    return lax.reduce_max(x, (0,)) if x.ndim else x


def _lane0(x):
    return lax.squeeze(lax.slice(x, (0,), (1,)), (0,))


def _ds16(j):
    return pl.ds(pl.multiple_of(j * L, L), L)


def _sc_body(scores_hbm, idx_hbm, val_hbm, data, hist, ck_a, ci_a, ck_b, ci_b,
             offs, stage_v, ncnt):
    cid = lax.axis_index("c")
    sid = lax.axis_index("s")
    wid = sid * 2 + cid
    iota = _iota()
    ones = jnp.ones((L,), jnp.int32)
    zeros = jnp.zeros((L,), jnp.int32)

    def zero_hist():
        def z(j, _):
            hist[_ds16(j)] = zeros
            return 0
        lax.fori_loop(0, NB // L, z, 0, unroll=8)

    def hist_range(nv):
        def h(j, _):
            v = data[_ds16(j)]
            d = lax.shift_right_arithmetic(_keys_from_f32(v), 21) + jnp.int32(1024)
            cnt, last = plsc.scan_count(d)
            plsc.addupdate_scatter(hist, [d], cnt, mask=last)
            return 0
        lax.fori_loop(0, nv, h, 0, unroll=4)

    def find_bucket(target):
        def s(i, carry):
            bkt, run, found = carry
            c = (NB // L - 1) - i
            tot = hist[_ds16(c)]
            rv = lax.rev(tot, (0,))
            cs = plsc.cumsum(rv) + run
            ge = cs >= target
            any_ = jnp.any(ge)
            j = _lane0(plsc.all_reduce_ffs(ge))
            bcand = c * L + (L - 1) - j
            bkt = jnp.where(jnp.logical_or(found, jnp.logical_not(any_)), bkt, bcand)
            found = jnp.logical_or(found, any_)
            return bkt, _splat_last(cs), found
        bkt, _, _ = lax.fori_loop(
            0, NB // L, s, (jnp.int32(0), jnp.zeros((L,), jnp.int32),
                            jnp.bool_(False)),
            unroll=2)
        return bkt

    def compact(kthr):
        def c(j, carry):
            offm1, idxv = carry
            key = _keys_from_f32(data[_ds16(j)])
            sel = key >= kthr
            pc = plsc.cumsum(sel.astype(jnp.int32))
            pos = offm1 + pc
            ok = jnp.logical_and(sel, pos < CAP)
            plsc.store_scatter(ck_a, [pos], key, mask=ok)
            plsc.store_scatter(ci_a, [pos], idxv, mask=ok)
            return (offm1 + _splat_last(pc), idxv + jnp.int32(L))
        offm1, _ = lax.fori_loop(0, NV_DATA, c, (zeros - 1, iota), unroll=4)
        return _lane0(offm1) + 1

    def row_body(t, _carry):
        row = wid * ROWS_PER_W + t
        pltpu.sync_copy(scores_hbm.at[row], data)

        zero_hist()
        hist_range(NV_SAMPLE)
        bkt = find_bucket(jnp.int32(M_SAMPLE))
        kthr = lax.shift_left(bkt - jnp.int32(1024), 21)
        raw = compact(kthr)
        ncnt[0] = raw

        @pl.when(jnp.logical_or(raw < K, raw > CAP))
        def _fallback():
            zero_hist()
            hist_range(NV_DATA)
            bkt2 = find_bucket(jnp.int32(K))
            kthr2 = lax.shift_left(bkt2 - jnp.int32(1024), 21)
            ncnt[0] = compact(kthr2)

        n_cand = jnp.minimum(ncnt[0], jnp.int32(CAP))

        minkey = jnp.full((L,), -0x80000000, jnp.int32)
        for m in range(4):
            pp = n_cand + iota + (m * L)
            plsc.store_scatter(ck_a, [pp], minkey, mask=pp < CAP)
            plsc.store_scatter(ci_a, [pp], zeros, mask=pp < CAP)
        ng = (n_cand + (4 * L - 1)) // (4 * L)

        for p, (shift, nbp, top) in enumerate(((0, 2048, False),
                                               (11, 2048, False),
                                               (22, 1024, True))):
            src_k, src_i = (ck_a, ci_a) if p % 2 == 0 else (ck_b, ci_b)
            dst_k, dst_i = (ck_b, ci_b) if p % 2 == 0 else (ck_a, ci_a)

            def digit(key):
                if top:
                    return lax.shift_right_arithmetic(key, shift) + jnp.int32(nbp // 2)
                return lax.shift_right_logical(key, shift) & jnp.int32(0x7FF)

            def zero_offs(j, _):
                offs[_ds16(j)] = zeros
                return 0
            lax.fori_loop(0, nbp // L, zero_offs, 0, unroll=8)

            def sort_hist_step(g, _):
                for u in range(4):
                    j = g * 4 + u
                    d = digit(src_k[_ds16(j)])
                    cnt, last = plsc.scan_count(d)
                    plsc.addupdate_scatter(offs, [d], cnt, mask=last)
                return 0
            lax.fori_loop(0, ng, sort_hist_step, 0)

            def suffix_step(i, run):
                c = (nbp // L - 1) - i
                v = offs[_ds16(c)]
                rv = lax.rev(v, (0,))
                cs = plsc.cumsum(rv)
                ex = cs - rv + run
                offs[_ds16(c)] = lax.rev(ex, (0,))
                return _splat_last(cs) + run
            lax.fori_loop(0, nbp // L, suffix_step, zeros, unroll=2)

            def permute_step(g, _):
                for u in range(4):
                    j = g * 4 + u
                    key = src_k[_ds16(j)]
                    idxv = src_i[_ds16(j)]
                    d = digit(key)
                    cnt, last = plsc.scan_count(d)
                    base = plsc.load_gather(offs, [d])
                    pos = base + cnt - 1
                    plsc.store_scatter(dst_k, [pos], key)
                    plsc.store_scatter(dst_i, [pos], idxv)
                    plsc.addupdate_scatter(offs, [d], cnt, mask=last)
                return 0
            lax.fori_loop(0, ng, permute_step, 0)

        def out_step(j, _):
            key = ck_b[_ds16(j)]
            stage_v[_ds16(j)] = _vals_from_keys(key)
            return 0
        lax.fori_loop(0, K // L, out_step, 0, unroll=4)
        pltpu.sync_copy(ci_b.at[pl.ds(0, K)], idx_hbm.at[row])
        pltpu.sync_copy(stage_v, val_hbm.at[row])
        return 0

    lax.fori_loop(0, ROWS_PER_W, row_body, 0)


@jax.jit
def _topk_sc(flat_scores):
    mesh = plsc.VectorSubcoreMesh(core_axis_name="c", subcore_axis_name="s")
    f = pl.kernel(
        _sc_body,
        out_type=(
            jax.ShapeDtypeStruct((N_ROWS, K), jnp.int32),
            jax.ShapeDtypeStruct((N_ROWS, K), jnp.float32),
        ),
        mesh=mesh,
        compiler_params=pltpu.CompilerParams(needs_layout_passes=False),
        scratch_types=[
            pltpu.VMEM((T_KV,), jnp.float32),
            pltpu.VMEM((NB,), jnp.int32),
            pltpu.VMEM((CAP,), jnp.int32),
            pltpu.VMEM((CAP,), jnp.int32),
            pltpu.VMEM((CAP,), jnp.int32),
            pltpu.VMEM((CAP,), jnp.int32),
            pltpu.VMEM((NB,), jnp.int32),
            pltpu.VMEM((K,), jnp.float32),
            pltpu.SMEM((1,), jnp.int32),
        ],
    )
    return f(flat_scores)


def kernel(scores):
    B, T, T_kv = scores.shape
    flat = scores.reshape(B * T, T_kv)
    idx, vals = _topk_sc(flat)
    indices = idx.reshape(B, T, K)
    mask = (vals != -jnp.inf).reshape(B, T, K)
    return indices, mask, jnp.array(K, dtype=jnp.int32)
